# Initial kernel scaffold; baseline (speedup 1.0000x reference)
import jax
import jax.numpy as jnp
from jax.experimental import pallas as pl


def kernel(nodes, edges, senders, receivers, globals_, enc_node_W, enc_node_b, enc_edge_W, enc_edge_b, mlp_W1, mlp_b1, mlp_W2, mlp_b2, dec_node_W, dec_node_b, dec_edge_W, dec_edge_b):
    raise NotImplementedError("write your pallas kernel here")


# speedup vs baseline: 3.1711x; 3.1711x over previous
import functools

import jax
import jax.numpy as jnp
from jax import lax
from jax.experimental import pallas as pl
from jax.experimental.pallas import tpu as pltpu
from jax.experimental.pallas import tpu_sc as plsc

_NS = 16
_WIN = 80
_N_PAD = 10240
_STRIPE = _N_PAD // _NS


def _sc_scatter_body(edges_hbm, sidx_hbm, ridx_hbm, zrows_hbm, out_hbm,
                     edges_v, si_w, ri_w, stage_v, acc_s, acc_r):
    s = lax.axis_index("s")
    eh = edges_v.shape[0]
    ew = 2 * eh
    kh = eh // _WIN
    pltpu.sync_copy(zrows_hbm, acc_s.at[pl.ds(s * _STRIPE, _STRIPE)])
    pltpu.sync_copy(zrows_hbm, acc_r.at[pl.ds(s * _STRIPE, _STRIPE)])
    plsc.subcore_barrier()

    for g in range(2):
        pltpu.sync_copy(edges_hbm.at[pl.ds(s * ew + g * eh, eh)], edges_v)

        def win(j, carry):
            base = s * ew + g * eh + j * _WIN
            pltpu.sync_copy(sidx_hbm.at[pl.ds(base, _WIN)], si_w)
            pltpu.sync_copy(ridx_hbm.at[pl.ds(base, _WIN)], ri_w)
            rows = edges_v.at[pl.ds(j * _WIN, _WIN)]
            pltpu.sync_copy(rows, acc_s.at[si_w], add=True)
            pltpu.sync_copy(rows, acc_r.at[ri_w], add=True)
            return carry

        lax.fori_loop(0, kh, win, 0)
    plsc.subcore_barrier()
    pltpu.sync_copy(acc_s.at[pl.ds(s * _STRIPE, _STRIPE)], stage_v)
    pltpu.sync_copy(stage_v, out_hbm.at[0, pl.ds(s * _STRIPE, _STRIPE)])
    pltpu.sync_copy(acc_r.at[pl.ds(s * _STRIPE, _STRIPE)], stage_v)
    pltpu.sync_copy(stage_v, out_hbm.at[1, pl.ds(s * _STRIPE, _STRIPE)])


def _sc_scatter(edges8, sidx, ridx):
    e = edges8.shape[0]
    ew = e // _NS
    zrows = jnp.zeros((_STRIPE, 8), jnp.float32)
    mesh = plsc.VectorSubcoreMesh(core_axis_name="c", subcore_axis_name="s")
    fn = functools.partial(
        pl.kernel,
        mesh=mesh,
        out_type=jax.ShapeDtypeStruct((2, _N_PAD, 8), jnp.float32),
        compiler_params=pltpu.CompilerParams(use_tc_tiling_on_sc=False),
        scratch_types=[
            pltpu.VMEM((ew // 2, 8), jnp.float32),
            pltpu.VMEM((_WIN,), jnp.int32),
            pltpu.VMEM((_WIN,), jnp.int32),
            pltpu.VMEM((_STRIPE, 8), jnp.float32),
            pltpu.VMEM_SHARED((_N_PAD, 8), jnp.float32),
            pltpu.VMEM_SHARED((_N_PAD, 8), jnp.float32),
        ],
    )(_sc_scatter_body)
    return fn(edges8, sidx, ridx, zrows)


def _node_body(nodes_ref, parts_ref, wn_ref, bn_ref, we_ref, be_ref, w1_ref,
               b1_ref, w2_ref, b2_ref, wd_ref, bd_ref, g_ref, out_ref):
    n = nodes_ref.shape[0]
    f32 = jnp.float32
    seg_s = parts_ref[0, :n, 0:4]
    cnt_s = parts_ref[0, :n, 4:5]
    seg_r = parts_ref[1, :n, 0:4]
    cnt_r = parts_ref[1, :n, 4:5]
    w1 = w1_ref[...]
    a = jnp.dot(wn_ref[...], w1[0:10, :], preferred_element_type=f32)
    b = jnp.dot(we_ref[...], w1[10:20, :], preferred_element_type=f32)
    cmat = jnp.dot(we_ref[...], w1[20:30, :], preferred_element_type=f32)
    be_b = jnp.dot(be_ref[...], w1[10:20, :], preferred_element_type=f32)
    be_c = jnp.dot(be_ref[...], w1[20:30, :], preferred_element_type=f32)
    c0 = (jnp.dot(bn_ref[...], w1[0:10, :], preferred_element_type=f32)
          + jnp.dot(g_ref[...], w1[30:34, :], preferred_element_type=f32)
          + b1_ref[...])
    pre = (jnp.dot(nodes_ref[...], a, preferred_element_type=f32)
           + jnp.dot(seg_s, b, preferred_element_type=f32)
           + cnt_s * be_b
           + jnp.dot(seg_r, cmat, preferred_element_type=f32)
           + cnt_r * be_c
           + c0)
    h = jnp.maximum(pre, 0.0)
    w_out = jnp.dot(w2_ref[...], wd_ref[...], preferred_element_type=f32)
    c1 = jnp.dot(b2_ref[...], wd_ref[...], preferred_element_type=f32) + bd_ref[...]
    out_ref[...] = jnp.dot(h, w_out, preferred_element_type=f32) + c1


def _edge_body(e_ref, we_ref, be_ref, wd_ref, bd_ref, o_ref):
    f32 = jnp.float32
    w = jnp.dot(we_ref[...], wd_ref[...], preferred_element_type=f32)
    c2 = jnp.dot(be_ref[...], wd_ref[...], preferred_element_type=f32) + bd_ref[...]
    o_ref[...] = jnp.dot(e_ref[...], w, preferred_element_type=f32) + c2


def kernel(nodes, edges, senders, receivers, globals_,
           enc_node_W, enc_node_b, enc_edge_W, enc_edge_b,
           mlp_W1, mlp_b1, mlp_W2, mlp_b2,
           dec_node_W, dec_node_b, dec_edge_W, dec_edge_b):
    n = nodes.shape[0]
    e = edges.shape[0]
    f32 = jnp.float32
    sidx = senders.astype(jnp.int32)
    ridx = receivers.astype(jnp.int32)
    pad = jnp.concatenate((jnp.ones((e, 1), f32), jnp.zeros((e, 3), f32)),
                          axis=1)
    edges8 = jnp.concatenate((edges, pad), axis=1)

    parts = _sc_scatter(edges8, sidx, ridx)

    nodes_out = pl.pallas_call(
        _node_body,
        out_shape=jax.ShapeDtypeStruct((n, 1), f32),
    )(nodes, parts, enc_node_W, enc_node_b.reshape(1, -1), enc_edge_W,
      enc_edge_b.reshape(1, -1), mlp_W1, mlp_b1.reshape(1, -1), mlp_W2,
      mlp_b2.reshape(1, -1), dec_node_W, dec_node_b.reshape(1, -1), globals_)

    blk = 4000
    grid = e // blk
    edges_out = pl.pallas_call(
        _edge_body,
        grid=(grid,),
        in_specs=[
            pl.BlockSpec((blk, 4), lambda i: (i, 0)),
            pl.BlockSpec((4, 10), lambda i: (0, 0)),
            pl.BlockSpec((1, 10), lambda i: (0, 0)),
            pl.BlockSpec((10, 1), lambda i: (0, 0)),
            pl.BlockSpec((1, 1), lambda i: (0, 0)),
        ],
        out_specs=pl.BlockSpec((blk, 1), lambda i: (i, 0)),
        out_shape=jax.ShapeDtypeStruct((e, 1), f32),
    )(edges, enc_edge_W, enc_edge_b.reshape(1, -1), dec_edge_W,
      dec_edge_b.reshape(1, -1))

    return nodes_out, edges_out, globals_
